# baseline (speedup 1.0000x reference)
import functools
import math

import jax
import jax.numpy as jnp
import numpy as np
from jax.experimental import pallas as pl
from jax.experimental.pallas import tpu as pltpu

S, D = 2048, 2048
H, KVH, HD = 16, 4, 128
GQ = H // KVH
BLK = 64
GH = 128
I = 5504
EPS = 1e-6
THRESH = 1e-3
THETA = 10000.0
NB = S // BLK

_F32 = jnp.float32


def _rms_matmul_body(x_ref, w_ref, b_ref, g_ref, o_ref):
    x = x_ref[...]
    var = jnp.mean(x * x, axis=-1, keepdims=True)
    xn = (x * jax.lax.rsqrt(var + EPS)) * g_ref[...]
    o_ref[...] = jnp.dot(xn, w_ref[...], preferred_element_type=_F32) + b_ref[...]


def _rms_matmul(x, w, b, g, bm, bn):
    m, k = x.shape
    n = w.shape[1]
    grid = (pl.cdiv(n, bn), pl.cdiv(m, bm))
    return pl.pallas_call(
        _rms_matmul_body,
        grid=grid,
        in_specs=[
            pl.BlockSpec((bm, k), lambda nn, mm: (mm, 0)),
            pl.BlockSpec((k, bn), lambda nn, mm: (0, nn)),
            pl.BlockSpec((1, bn), lambda nn, mm: (0, nn)),
            pl.BlockSpec((1, k), lambda nn, mm: (0, 0)),
        ],
        out_specs=pl.BlockSpec((bm, bn), lambda nn, mm: (mm, nn)),
        out_shape=jax.ShapeDtypeStruct((m, n), _F32),
    )(x, w, b.reshape(1, n), g.reshape(1, k))


def _matmul_res_body(x_ref, w_ref, r_ref, o_ref):
    o_ref[...] = r_ref[...] + jnp.dot(
        x_ref[...], w_ref[...], preferred_element_type=_F32)


def _matmul_res(x, w, r, bm, bn):
    m, k = x.shape
    n = w.shape[1]
    grid = (pl.cdiv(n, bn), pl.cdiv(m, bm))
    return pl.pallas_call(
        _matmul_res_body,
        grid=grid,
        in_specs=[
            pl.BlockSpec((bm, k), lambda nn, mm: (mm, 0)),
            pl.BlockSpec((k, bn), lambda nn, mm: (0, nn)),
            pl.BlockSpec((bm, bn), lambda nn, mm: (mm, nn)),
        ],
        out_specs=pl.BlockSpec((bm, bn), lambda nn, mm: (mm, nn)),
        out_shape=jax.ShapeDtypeStruct((m, n), _F32),
    )(x, w, r)


def _mlp1_body(x_ref, gw_ref, uw_ref, g_ref, o_ref):
    x = x_ref[...]
    var = jnp.mean(x * x, axis=-1, keepdims=True)
    xn = (x * jax.lax.rsqrt(var + EPS)) * g_ref[...]
    a = jnp.dot(xn, gw_ref[...], preferred_element_type=_F32)
    u = jnp.dot(xn, uw_ref[...], preferred_element_type=_F32)
    o_ref[...] = (a * jax.nn.sigmoid(a)) * u


def _mlp1(x, gw, uw, g, bm, bn):
    m, k = x.shape
    n = gw.shape[1]
    grid = (pl.cdiv(n, bn), pl.cdiv(m, bm))
    return pl.pallas_call(
        _mlp1_body,
        grid=grid,
        in_specs=[
            pl.BlockSpec((bm, k), lambda nn, mm: (mm, 0)),
            pl.BlockSpec((k, bn), lambda nn, mm: (0, nn)),
            pl.BlockSpec((k, bn), lambda nn, mm: (0, nn)),
            pl.BlockSpec((1, k), lambda nn, mm: (0, 0)),
        ],
        out_specs=pl.BlockSpec((bm, bn), lambda nn, mm: (mm, nn)),
        out_shape=jax.ShapeDtypeStruct((m, n), _F32),
    )(x, gw, uw, g.reshape(1, k))


BQ = 256
BQB = BQ // BLK
BKV = 512
MQ = S // BQ
NJ = S // BKV
_SCALE = 1.0 / math.sqrt(HD)


def _flash_body(q_ref, k_ref, v_ref, b_ref, o_ref, acc_ref, mx_ref, l_ref):
    mi = pl.program_id(2)
    jj = pl.program_id(3)

    @pl.when(2 * jj <= mi)
    def _run():
        @pl.when(jj == 0)
        def _init():
            mx_ref[...] = jnp.full_like(mx_ref, -1e30)
            l_ref[...] = jnp.zeros_like(l_ref)
            acc_ref[...] = jnp.zeros_like(acc_ref)

        q = q_ref[0, 0]
        k = k_ref[0, pl.ds(jj * BKV, BKV), :]
        s = jax.lax.dot_general(
            q, k, (((1,), (1,)), ((), ())),
            preferred_element_type=_F32) * _SCALE
        b = b_ref[0, 0, 0]
        s = (s.reshape(BQB, BLK, BKV) + b[:, None, :]).reshape(BQ, BKV)
        rows = mi * BQ + jax.lax.broadcasted_iota(jnp.int32, (BQ, BKV), 0)
        cols = jj * BKV + jax.lax.broadcasted_iota(jnp.int32, (BQ, BKV), 1)
        s = jnp.where(cols <= rows, s, -1e9)

        m_prev = mx_ref[...]
        m_new = jnp.maximum(m_prev, jnp.max(s, axis=-1, keepdims=True))
        p = jnp.exp(s - m_new)
        alpha = jnp.exp(m_prev - m_new)
        l_ref[...] = l_ref[...] * alpha + jnp.sum(p, axis=-1, keepdims=True)
        vv = v_ref[0, pl.ds(jj * BKV, BKV), :]
        acc_ref[...] = acc_ref[...] * alpha + jnp.dot(
            p, vv, preferred_element_type=_F32)
        mx_ref[...] = m_new

        @pl.when(jj == mi // 2)
        def _done():
            o_ref[0, 0] = acc_ref[...] / l_ref[...]


def _flash(qr, kr, vt, bias):
    grid = (KVH, GQ, MQ, NJ)
    return pl.pallas_call(
        _flash_body,
        grid=grid,
        in_specs=[
            pl.BlockSpec((1, 1, BQ, HD), lambda g, h, m, j: (g, h, m, 0)),
            pl.BlockSpec((1, S, HD), lambda g, h, m, j: (g, 0, 0)),
            pl.BlockSpec((1, S, HD), lambda g, h, m, j: (g, 0, 0)),
            pl.BlockSpec((1, 1, 1, BQB, BKV), lambda g, h, m, j: (g, h, m, 0, j)),
        ],
        out_specs=pl.BlockSpec((1, 1, BQ, HD), lambda g, h, m, j: (g, h, m, 0)),
        out_shape=jax.ShapeDtypeStruct((KVH, GQ, S, HD), _F32),
        scratch_shapes=[
            pltpu.VMEM((BQ, HD), _F32),
            pltpu.VMEM((BQ, 1), _F32),
            pltpu.VMEM((BQ, 1), _F32),
        ],
    )(qr, kr, vt, bias)


def _gate_bias(q, k, gq_w, gk_w):
    q_pool = q.reshape(NB, BLK, H, HD).mean(axis=1)
    k_pool = k.reshape(NB, BLK, KVH, HD).mean(axis=1)
    qg = jnp.einsum('nhd,dg->nhg', q_pool, gq_w)
    kg = jnp.einsum('nhd,dg->nhg', k_pool, gk_w)
    kg = jnp.repeat(kg, GQ, axis=1)
    logits = jnp.einsum('qhg,khg->hqk', qg, kg) / np.sqrt(GH)
    blk_causal = jnp.tril(jnp.ones((NB, NB), dtype=bool))
    logits = jnp.where(blk_causal[None], logits, -1e9)
    score = jax.nn.softmax(logits, axis=-1)
    diag = jnp.eye(NB, dtype=bool)
    mask = ((score >= THRESH) | diag[None]) & blk_causal[None]
    bias = jnp.where(mask, 0.0, -1e9).astype(_F32)
    bias_tok = jnp.repeat(bias, BLK, axis=2)
    return bias_tok


def _rope_tables(position_ids):
    inv_freq = 1.0 / (THETA ** (jnp.arange(0, HD, 2, dtype=_F32) / HD))
    freqs = position_ids[0].astype(_F32)[:, None] * inv_freq[None, :]
    emb = jnp.concatenate([freqs, freqs], axis=-1)
    return jnp.cos(emb), jnp.sin(emb)


def _rope(x, cos, sin):
    x1, x2 = jnp.split(x, 2, axis=-1)
    rot = jnp.concatenate([-x2, x1], axis=-1)
    return x * cos[:, None, :] + rot * sin[:, None, :]


def kernel(hidden_states, position_ids, ln1_w, q_w, q_b, k_w, k_b, v_w, v_b,
           o_w, gq_w, gk_w, ln2_w, gate_w, up_w, down_w):
    hs = hidden_states.reshape(S, D)

    wqkv = jnp.concatenate([q_w, k_w, v_w], axis=1)
    bqkv = jnp.concatenate([q_b, k_b, v_b], axis=0)
    qkv = _rms_matmul(hs, wqkv, bqkv, ln1_w, bm=1024, bn=1024)

    q = qkv[:, :H * HD].reshape(S, H, HD)
    k = qkv[:, H * HD:(H + KVH) * HD].reshape(S, KVH, HD)
    v = qkv[:, (H + KVH) * HD:].reshape(S, KVH, HD)

    bias_tok = _gate_bias(q, k, gq_w, gk_w)

    cos, sin = _rope_tables(position_ids)
    qr = _rope(q, cos, sin).transpose(1, 0, 2).reshape(KVH, GQ, S, HD)
    kr = _rope(k, cos, sin).transpose(1, 0, 2)
    vt = v.transpose(1, 0, 2)
    bias4 = bias_tok.reshape(KVH, GQ, MQ, BQB, S)

    attn = _flash(qr, kr, vt, bias4)
    attn2 = attn.transpose(2, 0, 1, 3).reshape(S, H * HD)

    hidden = _matmul_res(attn2, o_w, hs, bm=512, bn=1024)

    mlp_mid = _mlp1(hidden, gate_w, up_w, ln2_w, bm=1024, bn=512)
    out = _matmul_res(mlp_mid, down_w, hidden, bm=512, bn=512)
    return out.reshape(1, S, D)


# speedup vs baseline: 1.0042x; 1.0042x over previous
import functools
import math

import jax
import jax.numpy as jnp
import numpy as np
from jax.experimental import pallas as pl
from jax.experimental.pallas import tpu as pltpu

S, D = 2048, 2048
H, KVH, HD = 16, 4, 128
GQ = H // KVH
BLK = 64
GH = 128
I = 5504
EPS = 1e-6
THRESH = 1e-3
THETA = 10000.0
NB = S // BLK

_F32 = jnp.float32
_BF16 = jnp.bfloat16


def _rms_matmul_body(x_ref, w_ref, b_ref, g_ref, o_ref):
    x = x_ref[...]
    var = jnp.mean(x * x, axis=-1, keepdims=True)
    xn = ((x * jax.lax.rsqrt(var + EPS)) * g_ref[...]).astype(_BF16)
    o_ref[...] = jnp.dot(xn, w_ref[...], preferred_element_type=_F32) + b_ref[...]


def _rms_matmul(x, w, b, g, bm, bn):
    m, k = x.shape
    n = w.shape[1]
    grid = (pl.cdiv(n, bn), pl.cdiv(m, bm))
    return pl.pallas_call(
        _rms_matmul_body,
        grid=grid,
        in_specs=[
            pl.BlockSpec((bm, k), lambda nn, mm: (mm, 0)),
            pl.BlockSpec((k, bn), lambda nn, mm: (0, nn)),
            pl.BlockSpec((1, bn), lambda nn, mm: (0, nn)),
            pl.BlockSpec((1, k), lambda nn, mm: (0, 0)),
        ],
        out_specs=pl.BlockSpec((bm, bn), lambda nn, mm: (mm, nn)),
        out_shape=jax.ShapeDtypeStruct((m, n), _F32),
    )(x, w, b.reshape(1, n), g.reshape(1, k))


def _matmul_res_body(x_ref, w_ref, r_ref, o_ref):
    o_ref[...] = r_ref[...] + jnp.dot(
        x_ref[...], w_ref[...], preferred_element_type=_F32)


def _matmul_res(x, w, r, bm, bn):
    m, k = x.shape
    n = w.shape[1]
    grid = (pl.cdiv(n, bn), pl.cdiv(m, bm))
    return pl.pallas_call(
        _matmul_res_body,
        grid=grid,
        in_specs=[
            pl.BlockSpec((bm, k), lambda nn, mm: (mm, 0)),
            pl.BlockSpec((k, bn), lambda nn, mm: (0, nn)),
            pl.BlockSpec((bm, bn), lambda nn, mm: (mm, nn)),
        ],
        out_specs=pl.BlockSpec((bm, bn), lambda nn, mm: (mm, nn)),
        out_shape=jax.ShapeDtypeStruct((m, n), _F32),
    )(x, w, r)


def _mlp1_body(x_ref, gw_ref, uw_ref, g_ref, o_ref):
    x = x_ref[...]
    var = jnp.mean(x * x, axis=-1, keepdims=True)
    xn = ((x * jax.lax.rsqrt(var + EPS)) * g_ref[...]).astype(_BF16)
    a = jnp.dot(xn, gw_ref[...], preferred_element_type=_F32)
    u = jnp.dot(xn, uw_ref[...], preferred_element_type=_F32)
    o_ref[...] = ((a * jax.nn.sigmoid(a)) * u).astype(_BF16)


def _mlp1(x, gw, uw, g, bm, bn):
    m, k = x.shape
    n = gw.shape[1]
    grid = (pl.cdiv(n, bn), pl.cdiv(m, bm))
    return pl.pallas_call(
        _mlp1_body,
        grid=grid,
        in_specs=[
            pl.BlockSpec((bm, k), lambda nn, mm: (mm, 0)),
            pl.BlockSpec((k, bn), lambda nn, mm: (0, nn)),
            pl.BlockSpec((k, bn), lambda nn, mm: (0, nn)),
            pl.BlockSpec((1, k), lambda nn, mm: (0, 0)),
        ],
        out_specs=pl.BlockSpec((bm, bn), lambda nn, mm: (mm, nn)),
        out_shape=jax.ShapeDtypeStruct((m, n), _BF16),
    )(x, gw, uw, g.reshape(1, k))


BQ = 256
BQB = BQ // BLK
BKV = 512
MQ = S // BQ
NJ = S // BKV
_SCALE = 1.0 / math.sqrt(HD)


def _flash_body(q_ref, k_ref, v_ref, b_ref, o_ref, acc_ref, mx_ref, l_ref):
    mi = pl.program_id(2)
    jj = pl.program_id(3)

    @pl.when(2 * jj <= mi)
    def _run():
        @pl.when(jj == 0)
        def _init():
            mx_ref[...] = jnp.full_like(mx_ref, -1e30)
            l_ref[...] = jnp.zeros_like(l_ref)
            acc_ref[...] = jnp.zeros_like(acc_ref)

        q = q_ref[...]
        k = k_ref[pl.ds(jj * BKV, BKV), :]
        s = jax.lax.dot_general(
            q, k, (((1,), (1,)), ((), ())),
            preferred_element_type=_F32) * _SCALE
        b = b_ref[0, 0, 0]
        s = (s.reshape(BQB, BLK, BKV) + b[:, None, :]).reshape(BQ, BKV)
        rows = mi * BQ + jax.lax.broadcasted_iota(jnp.int32, (BQ, BKV), 0)
        cols = jj * BKV + jax.lax.broadcasted_iota(jnp.int32, (BQ, BKV), 1)
        s = jnp.where(cols <= rows, s, -1e9)

        m_prev = mx_ref[...]
        m_new = jnp.maximum(m_prev, jnp.max(s, axis=-1, keepdims=True))
        p = jnp.exp(s - m_new)
        alpha = jnp.exp(m_prev - m_new)
        l_ref[...] = l_ref[...] * alpha + jnp.sum(p, axis=-1, keepdims=True)
        vv = v_ref[pl.ds(jj * BKV, BKV), :]
        acc_ref[...] = acc_ref[...] * alpha + jnp.dot(
            p.astype(_BF16), vv, preferred_element_type=_F32)
        mx_ref[...] = m_new

        @pl.when(jj == mi // 2)
        def _done():
            o_ref[...] = acc_ref[...] / l_ref[...]


def _flash(qf, kf, vf, bias):
    grid = (KVH, GQ, MQ, NJ)
    return pl.pallas_call(
        _flash_body,
        grid=grid,
        in_specs=[
            pl.BlockSpec((BQ, HD), lambda g, h, m, j: (m, g * GQ + h)),
            pl.BlockSpec((S, HD), lambda g, h, m, j: (0, g)),
            pl.BlockSpec((S, HD), lambda g, h, m, j: (0, g)),
            pl.BlockSpec((1, 1, 1, BQB, BKV), lambda g, h, m, j: (g, h, m, 0, j)),
        ],
        out_specs=pl.BlockSpec((BQ, HD), lambda g, h, m, j: (m, g * GQ + h)),
        out_shape=jax.ShapeDtypeStruct((S, H * HD), _F32),
        scratch_shapes=[
            pltpu.VMEM((BQ, HD), _F32),
            pltpu.VMEM((BQ, 1), _F32),
            pltpu.VMEM((BQ, 1), _F32),
        ],
    )(qf, kf, vf, bias)


def _gate_bias(q, k, gq_w, gk_w):
    q_pool = q.reshape(NB, BLK, H, HD).mean(axis=1)
    k_pool = k.reshape(NB, BLK, KVH, HD).mean(axis=1)
    qg = jnp.einsum('nhd,dg->nhg', q_pool, gq_w)
    kg = jnp.einsum('nhd,dg->nhg', k_pool, gk_w)
    kg = jnp.repeat(kg, GQ, axis=1)
    logits = jnp.einsum('qhg,khg->hqk', qg, kg) / np.sqrt(GH)
    blk_causal = jnp.tril(jnp.ones((NB, NB), dtype=bool))
    logits = jnp.where(blk_causal[None], logits, -1e9)
    score = jax.nn.softmax(logits, axis=-1)
    diag = jnp.eye(NB, dtype=bool)
    mask = ((score >= THRESH) | diag[None]) & blk_causal[None]
    bias = jnp.where(mask, 0.0, -1e9).astype(_F32)
    bias_tok = jnp.repeat(bias, BLK, axis=2)
    return bias_tok


def _rope_tables(position_ids):
    inv_freq = 1.0 / (THETA ** (jnp.arange(0, HD, 2, dtype=_F32) / HD))
    freqs = position_ids[0].astype(_F32)[:, None] * inv_freq[None, :]
    emb = jnp.concatenate([freqs, freqs], axis=-1)
    return jnp.cos(emb), jnp.sin(emb)


def _rope(x, cos, sin):
    x1, x2 = jnp.split(x, 2, axis=-1)
    rot = jnp.concatenate([-x2, x1], axis=-1)
    return x * cos[:, None, :] + rot * sin[:, None, :]


def kernel(hidden_states, position_ids, ln1_w, q_w, q_b, k_w, k_b, v_w, v_b,
           o_w, gq_w, gk_w, ln2_w, gate_w, up_w, down_w):
    hs = hidden_states.reshape(S, D)

    wqkv = jnp.concatenate([q_w, k_w, v_w], axis=1).astype(_BF16)
    bqkv = jnp.concatenate([q_b, k_b, v_b], axis=0)
    qkv = _rms_matmul(hs, wqkv, bqkv, ln1_w, bm=1024, bn=1024)

    q = qkv[:, :H * HD].reshape(S, H, HD)
    k = qkv[:, H * HD:(H + KVH) * HD].reshape(S, KVH, HD)
    v = qkv[:, (H + KVH) * HD:]

    bias_tok = _gate_bias(q, k, gq_w, gk_w)

    cos, sin = _rope_tables(position_ids)
    qf = _rope(q, cos, sin).reshape(S, H * HD).astype(_BF16)
    kf = _rope(k, cos, sin).reshape(S, KVH * HD).astype(_BF16)
    vf = v.astype(_BF16)
    bias4 = bias_tok.reshape(KVH, GQ, MQ, BQB, S)

    attn2 = _flash(qf, kf, vf, bias4)

    hidden = _matmul_res(attn2.astype(_BF16), o_w.astype(_BF16), hs,
                         bm=512, bn=1024)

    mlp_mid = _mlp1(hidden, gate_w.astype(_BF16), up_w.astype(_BF16), ln2_w,
                    bm=1024, bn=512)
    out = _matmul_res(mlp_mid, down_w.astype(_BF16), hidden, bm=512, bn=512)
    return out.reshape(1, S, D)


# speedup vs baseline: 1.4690x; 1.4629x over previous
import functools
import math

import jax
import jax.numpy as jnp
import numpy as np
from jax.experimental import pallas as pl
from jax.experimental.pallas import tpu as pltpu

S, D = 2048, 2048
H, KVH, HD = 16, 4, 128
GQ = H // KVH
BLK = 64
GH = 128
I = 5504
EPS = 1e-6
THRESH = 1e-3
THETA = 10000.0
NB = S // BLK

_F32 = jnp.float32
_BF16 = jnp.bfloat16


def _rmsnorm_body(x_ref, g_ref, o_ref):
    x = x_ref[...]
    var = jnp.mean(x * x, axis=-1, keepdims=True)
    o_ref[...] = ((x * jax.lax.rsqrt(var + EPS)) * g_ref[...]).astype(_BF16)


def _rmsnorm(x, g, bm=512):
    m, k = x.shape
    return pl.pallas_call(
        _rmsnorm_body,
        grid=(m // bm,),
        in_specs=[
            pl.BlockSpec((bm, k), lambda mm: (mm, 0)),
            pl.BlockSpec((1, k), lambda mm: (0, 0)),
        ],
        out_specs=pl.BlockSpec((bm, k), lambda mm: (mm, 0)),
        out_shape=jax.ShapeDtypeStruct((m, k), _BF16),
    )(x, g.reshape(1, k))


_QKV_BM = 512
_PB = _QKV_BM // BLK


def _qkv_body(x_ref, qw_ref, kw_ref, vw_ref, b_ref, o_ref, p_ref):
    x = x_ref[...]
    qw = qw_ref[...].astype(_BF16)
    kw = kw_ref[...].astype(_BF16)
    vw = vw_ref[...].astype(_BF16)
    oq = jnp.dot(x, qw, preferred_element_type=_F32)
    ok = jnp.dot(x, kw, preferred_element_type=_F32)
    ov = jnp.dot(x, vw, preferred_element_type=_F32)
    out = jnp.concatenate([oq, ok, ov], axis=-1) + b_ref[...]
    o_ref[...] = out
    p_ref[...] = out.reshape(_PB, BLK, (H + 2 * KVH) * HD).sum(axis=1)


def _qkv(xn, qw, kw, vw, b):
    n_all = (H + 2 * KVH) * HD
    return pl.pallas_call(
        _qkv_body,
        grid=(S // _QKV_BM,),
        in_specs=[
            pl.BlockSpec((_QKV_BM, D), lambda mm: (mm, 0)),
            pl.BlockSpec((D, H * HD), lambda mm: (0, 0)),
            pl.BlockSpec((D, KVH * HD), lambda mm: (0, 0)),
            pl.BlockSpec((D, KVH * HD), lambda mm: (0, 0)),
            pl.BlockSpec((1, n_all), lambda mm: (0, 0)),
        ],
        out_specs=[
            pl.BlockSpec((_QKV_BM, n_all), lambda mm: (mm, 0)),
            pl.BlockSpec((_PB, n_all), lambda mm: (mm, 0)),
        ],
        out_shape=[
            jax.ShapeDtypeStruct((S, n_all), _F32),
            jax.ShapeDtypeStruct((NB, n_all), _F32),
        ],
    )(xn, qw, kw, vw, b.reshape(1, n_all))


def _matmul_res_body(x_ref, w_ref, r_ref, o_ref):
    w = w_ref[...].astype(_BF16)
    o_ref[...] = r_ref[...] + jnp.dot(
        x_ref[...], w, preferred_element_type=_F32)


def _matmul_res(x, w, r, bm, bn):
    m, k = x.shape
    n = w.shape[1]
    grid = (pl.cdiv(n, bn), pl.cdiv(m, bm))
    return pl.pallas_call(
        _matmul_res_body,
        grid=grid,
        in_specs=[
            pl.BlockSpec((bm, k), lambda nn, mm: (mm, 0)),
            pl.BlockSpec((k, bn), lambda nn, mm: (0, nn)),
            pl.BlockSpec((bm, bn), lambda nn, mm: (mm, nn)),
        ],
        out_specs=pl.BlockSpec((bm, bn), lambda nn, mm: (mm, nn)),
        out_shape=jax.ShapeDtypeStruct((m, n), _F32),
    )(x, w, r)


def _mlp1_body(x_ref, gw_ref, uw_ref, o_ref):
    x = x_ref[...]
    a = jnp.dot(x, gw_ref[...].astype(_BF16), preferred_element_type=_F32)
    u = jnp.dot(x, uw_ref[...].astype(_BF16), preferred_element_type=_F32)
    o_ref[...] = ((a * jax.nn.sigmoid(a)) * u).astype(_BF16)


def _mlp1(x, gw, uw, bm, bn):
    m, k = x.shape
    n = gw.shape[1]
    grid = (pl.cdiv(n, bn), pl.cdiv(m, bm))
    return pl.pallas_call(
        _mlp1_body,
        grid=grid,
        in_specs=[
            pl.BlockSpec((bm, k), lambda nn, mm: (mm, 0)),
            pl.BlockSpec((k, bn), lambda nn, mm: (0, nn)),
            pl.BlockSpec((k, bn), lambda nn, mm: (0, nn)),
        ],
        out_specs=pl.BlockSpec((bm, bn), lambda nn, mm: (mm, nn)),
        out_shape=jax.ShapeDtypeStruct((m, n), _BF16),
    )(x, gw, uw)


BQ = 256
BQB = BQ // BLK
BKV = 512
MQ = S // BQ
_SCALE = 1.0 / math.sqrt(HD)
_NEG = -1e9


def _flash_body(q_ref, k_ref, v_ref, b_ref, o_ref):
    mi = pl.program_id(1)
    jlast = mi // 2

    for h in range(GQ):
        q = q_ref[:, h * HD:(h + 1) * HD]

        def chunk(jj, carry, causal):
            m_prev, l_prev, acc = carry
            kc = k_ref[pl.ds(jj * BKV, BKV), :]
            s = jax.lax.dot_general(
                q, kc, (((1,), (1,)), ((), ())),
                preferred_element_type=_F32) * _SCALE
            bc = b_ref[0, h, 0, :, pl.ds(jj * BKV, BKV)]
            s = (s.reshape(BQB, BLK, BKV) + bc[:, None, :]).reshape(BQ, BKV)
            if causal:
                rows = mi * BQ + jax.lax.broadcasted_iota(
                    jnp.int32, (BQ, BKV), 0)
                cols = jj * BKV + jax.lax.broadcasted_iota(
                    jnp.int32, (BQ, BKV), 1)
                s = jnp.where(cols <= rows, s, _NEG)
            m_new = jnp.maximum(m_prev, jnp.max(s, axis=-1, keepdims=True))
            p = jnp.exp(s - m_new)
            alpha = jnp.exp(m_prev - m_new)
            l_new = l_prev * alpha + jnp.sum(p, axis=-1, keepdims=True)
            vc = v_ref[pl.ds(jj * BKV, BKV), :]
            acc_new = acc * alpha + jnp.dot(
                p.astype(_BF16), vc, preferred_element_type=_F32)
            return m_new, l_new, acc_new

        init = (jnp.full((BQ, 1), -1e30, _F32),
                jnp.zeros((BQ, 1), _F32),
                jnp.zeros((BQ, HD), _F32))
        carry = jax.lax.fori_loop(
            0, jlast, lambda jj, c: chunk(jj, c, causal=False), init)
        _, l_fin, acc_fin = chunk(jlast, carry, causal=True)
        o_ref[:, h * HD:(h + 1) * HD] = (acc_fin / l_fin).astype(_BF16)


def _flash(qf, kf, vf, bias):
    grid = (KVH, MQ)
    return pl.pallas_call(
        _flash_body,
        grid=grid,
        in_specs=[
            pl.BlockSpec((BQ, GQ * HD), lambda g, m: (m, g)),
            pl.BlockSpec((S, HD), lambda g, m: (0, g)),
            pl.BlockSpec((S, HD), lambda g, m: (0, g)),
            pl.BlockSpec((1, GQ, 1, BQB, S), lambda g, m: (g, 0, m, 0, 0)),
        ],
        out_specs=pl.BlockSpec((BQ, GQ * HD), lambda g, m: (m, g)),
        out_shape=jax.ShapeDtypeStruct((S, H * HD), _BF16),
    )(qf, kf, vf, bias)


def _gate_bias(pooled):
    q_pool = pooled[:, :H * HD].reshape(NB, H, HD) / BLK
    k_pool = pooled[:, H * HD:(H + KVH) * HD].reshape(NB, KVH, HD) / BLK
    return q_pool, k_pool


def _gate_mask_bias(q_pool, k_pool, gq_w, gk_w):
    qg = jnp.einsum('nhd,dg->nhg', q_pool, gq_w,
                    precision=jax.lax.Precision.HIGHEST)
    kg = jnp.einsum('nhd,dg->nhg', k_pool, gk_w,
                    precision=jax.lax.Precision.HIGHEST)
    kg = jnp.repeat(kg, GQ, axis=1)
    logits = jnp.einsum('qhg,khg->hqk', qg, kg,
                        precision=jax.lax.Precision.HIGHEST) / np.sqrt(GH)
    blk_causal = jnp.tril(jnp.ones((NB, NB), dtype=bool))
    logits = jnp.where(blk_causal[None], logits, _NEG)
    score = jax.nn.softmax(logits, axis=-1)
    diag = jnp.eye(NB, dtype=bool)
    mask = ((score >= THRESH) | diag[None]) & blk_causal[None]
    bias = jnp.where(mask, 0.0, _NEG).astype(_F32)
    bias_tok = jnp.repeat(bias, BLK, axis=2)
    return bias_tok


def _rope_tables(position_ids):
    inv_freq = 1.0 / (THETA ** (jnp.arange(0, HD, 2, dtype=_F32) / HD))
    freqs = position_ids[0].astype(_F32)[:, None] * inv_freq[None, :]
    emb = jnp.concatenate([freqs, freqs], axis=-1)
    return jnp.cos(emb), jnp.sin(emb)


def _rope(x, cos, sin):
    x1, x2 = jnp.split(x, 2, axis=-1)
    rot = jnp.concatenate([-x2, x1], axis=-1)
    return x * cos[:, None, :] + rot * sin[:, None, :]


def kernel(hidden_states, position_ids, ln1_w, q_w, q_b, k_w, k_b, v_w, v_b,
           o_w, gq_w, gk_w, ln2_w, gate_w, up_w, down_w):
    hs = hidden_states.reshape(S, D)

    xn1 = _rmsnorm(hs, ln1_w)
    bqkv = jnp.concatenate([q_b, k_b, v_b], axis=0)
    qkv, pooled = _qkv(xn1, q_w, k_w, v_w, bqkv)

    q_pool, k_pool = _gate_bias(pooled)
    bias_tok = _gate_mask_bias(q_pool, k_pool, gq_w, gk_w)
    bias5 = bias_tok.reshape(KVH, GQ, MQ, BQB, S)

    q = qkv[:, :H * HD].reshape(S, H, HD)
    k = qkv[:, H * HD:(H + KVH) * HD].reshape(S, KVH, HD)
    v = qkv[:, (H + KVH) * HD:]

    cos, sin = _rope_tables(position_ids)
    qf = _rope(q, cos, sin).reshape(S, H * HD).astype(_BF16)
    kf = _rope(k, cos, sin).reshape(S, KVH * HD).astype(_BF16)
    vf = v.astype(_BF16)

    attn2 = _flash(qf, kf, vf, bias5)

    hidden = _matmul_res(attn2, o_w, hs, bm=512, bn=1024)

    xn2 = _rmsnorm(hidden, ln2_w)
    mlp_mid = _mlp1(xn2, gate_w, up_w, bm=1024, bn=512)
    out = _matmul_res(mlp_mid, down_w, hidden, bm=512, bn=512)
    return out.reshape(1, S, D)


# speedup vs baseline: 1.8376x; 1.2509x over previous
import functools
import math

import jax
import jax.numpy as jnp
import numpy as np
from jax.experimental import pallas as pl
from jax.experimental.pallas import tpu as pltpu

S, D = 2048, 2048
H, KVH, HD = 16, 4, 128
GQ = H // KVH
BLK = 64
GH = 128
I = 5504
EPS = 1e-6
THRESH = 1e-3
THETA = 10000.0
NB = S // BLK

_F32 = jnp.float32
_BF16 = jnp.bfloat16


def _rmsnorm_body(x_ref, g_ref, o_ref):
    x = x_ref[...]
    var = jnp.mean(x * x, axis=-1, keepdims=True)
    o_ref[...] = ((x * jax.lax.rsqrt(var + EPS)) * g_ref[...]).astype(_BF16)


def _rmsnorm(x, g, bm=512):
    m, k = x.shape
    return pl.pallas_call(
        _rmsnorm_body,
        grid=(m // bm,),
        in_specs=[
            pl.BlockSpec((bm, k), lambda mm: (mm, 0)),
            pl.BlockSpec((1, k), lambda mm: (0, 0)),
        ],
        out_specs=pl.BlockSpec((bm, k), lambda mm: (mm, 0)),
        out_shape=jax.ShapeDtypeStruct((m, k), _BF16),
    )(x, g.reshape(1, k))


_QKV_BM = 512
_PB = _QKV_BM // BLK


def _rope_piece(x, c, sn):
    x1 = x[:, :HD // 2]
    x2 = x[:, HD // 2:]
    rot = jnp.concatenate([-x2, x1], axis=-1)
    return x * c + rot * sn


def _qkv_body(x_ref, qw_ref, kw_ref, vw_ref, b_ref, cos_ref, sin_ref,
              o_ref, p_ref):
    x = x_ref[...]
    qw = qw_ref[...].astype(_BF16)
    kw = kw_ref[...].astype(_BF16)
    vw = vw_ref[...].astype(_BF16)
    oq = jnp.dot(x, qw, preferred_element_type=_F32)
    ok = jnp.dot(x, kw, preferred_element_type=_F32)
    ov = jnp.dot(x, vw, preferred_element_type=_F32)
    out = jnp.concatenate([oq, ok, ov], axis=-1) + b_ref[...]
    p_ref[...] = out.reshape(_PB, BLK, (H + 2 * KVH) * HD).sum(axis=1)
    c = cos_ref[...]
    sn = sin_ref[...]
    for hh in range(H + KVH):
        piece = out[:, hh * HD:(hh + 1) * HD]
        o_ref[:, hh * HD:(hh + 1) * HD] = _rope_piece(piece, c, sn).astype(_BF16)
    o_ref[:, (H + KVH) * HD:] = out[:, (H + KVH) * HD:].astype(_BF16)


def _qkv(xn, qw, kw, vw, b, cos, sin):
    n_all = (H + 2 * KVH) * HD
    return pl.pallas_call(
        _qkv_body,
        grid=(S // _QKV_BM,),
        in_specs=[
            pl.BlockSpec((_QKV_BM, D), lambda mm: (mm, 0)),
            pl.BlockSpec((D, H * HD), lambda mm: (0, 0)),
            pl.BlockSpec((D, KVH * HD), lambda mm: (0, 0)),
            pl.BlockSpec((D, KVH * HD), lambda mm: (0, 0)),
            pl.BlockSpec((1, n_all), lambda mm: (0, 0)),
            pl.BlockSpec((_QKV_BM, HD), lambda mm: (mm, 0)),
            pl.BlockSpec((_QKV_BM, HD), lambda mm: (mm, 0)),
        ],
        out_specs=[
            pl.BlockSpec((_QKV_BM, n_all), lambda mm: (mm, 0)),
            pl.BlockSpec((_PB, n_all), lambda mm: (mm, 0)),
        ],
        out_shape=[
            jax.ShapeDtypeStruct((S, n_all), _BF16),
            jax.ShapeDtypeStruct((NB, n_all), _F32),
        ],
    )(xn, qw, kw, vw, b.reshape(1, n_all), cos, sin)


def _matmul_res_body(x_ref, w_ref, r_ref, o_ref):
    w = w_ref[...].astype(_BF16)
    o_ref[...] = r_ref[...] + jnp.dot(
        x_ref[...], w, preferred_element_type=_F32)


def _matmul_res(x, w, r, bm, bn):
    m, k = x.shape
    n = w.shape[1]
    grid = (pl.cdiv(n, bn), pl.cdiv(m, bm))
    return pl.pallas_call(
        _matmul_res_body,
        grid=grid,
        in_specs=[
            pl.BlockSpec((bm, k), lambda nn, mm: (mm, 0)),
            pl.BlockSpec((k, bn), lambda nn, mm: (0, nn)),
            pl.BlockSpec((bm, bn), lambda nn, mm: (mm, nn)),
        ],
        out_specs=pl.BlockSpec((bm, bn), lambda nn, mm: (mm, nn)),
        out_shape=jax.ShapeDtypeStruct((m, n), _F32),
    )(x, w, r)


def _mlp1_body(x_ref, gw_ref, uw_ref, o_ref):
    x = x_ref[...]
    a = jnp.dot(x, gw_ref[...].astype(_BF16), preferred_element_type=_F32)
    u = jnp.dot(x, uw_ref[...].astype(_BF16), preferred_element_type=_F32)
    o_ref[...] = ((a * jax.nn.sigmoid(a)) * u).astype(_BF16)


def _mlp1(x, gw, uw, bm, bn):
    m, k = x.shape
    n = gw.shape[1]
    grid = (pl.cdiv(n, bn), pl.cdiv(m, bm))
    return pl.pallas_call(
        _mlp1_body,
        grid=grid,
        in_specs=[
            pl.BlockSpec((bm, k), lambda nn, mm: (mm, 0)),
            pl.BlockSpec((k, bn), lambda nn, mm: (0, nn)),
            pl.BlockSpec((k, bn), lambda nn, mm: (0, nn)),
        ],
        out_specs=pl.BlockSpec((bm, bn), lambda nn, mm: (mm, nn)),
        out_shape=jax.ShapeDtypeStruct((m, n), _BF16),
    )(x, gw, uw)


BQ = 256
BQB = BQ // BLK
BKV = 512
MQ = S // BQ
_SCALE = 1.0 / math.sqrt(HD)
_NEG = -1e9


def _flash_body(q_ref, k_ref, v_ref, b_ref, o_ref):
    mi = pl.program_id(1)
    jlast = mi // 2

    for h in range(GQ):
        q = q_ref[:, h * HD:(h + 1) * HD]

        def chunk(jj, carry, causal):
            m_prev, l_prev, acc = carry
            kc = k_ref[pl.ds(jj * BKV, BKV), :]
            s = jax.lax.dot_general(
                q, kc, (((1,), (1,)), ((), ())),
                preferred_element_type=_F32) * _SCALE
            bc = b_ref[0, h, 0, :, pl.ds(jj * BKV, BKV)]
            s = (s.reshape(BQB, BLK, BKV) + bc[:, None, :]).reshape(BQ, BKV)
            if causal:
                rows = mi * BQ + jax.lax.broadcasted_iota(
                    jnp.int32, (BQ, BKV), 0)
                cols = jj * BKV + jax.lax.broadcasted_iota(
                    jnp.int32, (BQ, BKV), 1)
                s = jnp.where(cols <= rows, s, _NEG)
            m_new = jnp.maximum(m_prev, jnp.max(s, axis=-1, keepdims=True))
            p = jnp.exp(s - m_new)
            alpha = jnp.exp(m_prev - m_new)
            l_new = l_prev * alpha + jnp.sum(p, axis=-1, keepdims=True)
            vc = v_ref[pl.ds(jj * BKV, BKV), :]
            acc_new = acc * alpha + jnp.dot(
                p.astype(_BF16), vc, preferred_element_type=_F32)
            return m_new, l_new, acc_new

        init = (jnp.full((BQ, 1), -1e30, _F32),
                jnp.zeros((BQ, 1), _F32),
                jnp.zeros((BQ, HD), _F32))
        carry = jax.lax.fori_loop(
            0, jlast, lambda jj, c: chunk(jj, c, causal=False), init)
        _, l_fin, acc_fin = chunk(jlast, carry, causal=True)
        o_ref[:, h * HD:(h + 1) * HD] = (acc_fin / l_fin).astype(_BF16)


def _flash(qkv, bias):
    grid = (KVH, MQ)
    return pl.pallas_call(
        _flash_body,
        grid=grid,
        in_specs=[
            pl.BlockSpec((BQ, GQ * HD), lambda g, m: (m, g)),
            pl.BlockSpec((S, HD), lambda g, m: (0, H + g)),
            pl.BlockSpec((S, HD), lambda g, m: (0, H + KVH + g)),
            pl.BlockSpec((1, GQ, 1, BQB, S), lambda g, m: (g, 0, m, 0, 0)),
        ],
        out_specs=pl.BlockSpec((BQ, GQ * HD), lambda g, m: (m, g)),
        out_shape=jax.ShapeDtypeStruct((S, H * HD), _BF16),
    )(qkv, qkv, qkv, bias)


def _gate_bias(pooled):
    q_pool = pooled[:, :H * HD].reshape(NB, H, HD) / BLK
    k_pool = pooled[:, H * HD:(H + KVH) * HD].reshape(NB, KVH, HD) / BLK
    return q_pool, k_pool


def _gate_mask_bias(q_pool, k_pool, gq_w, gk_w):
    qg = jnp.einsum('nhd,dg->nhg', q_pool, gq_w,
                    precision=jax.lax.Precision.HIGHEST)
    kg = jnp.einsum('nhd,dg->nhg', k_pool, gk_w,
                    precision=jax.lax.Precision.HIGHEST)
    kg = jnp.repeat(kg, GQ, axis=1)
    logits = jnp.einsum('qhg,khg->hqk', qg, kg,
                        precision=jax.lax.Precision.HIGHEST) / np.sqrt(GH)
    blk_causal = jnp.tril(jnp.ones((NB, NB), dtype=bool))
    logits = jnp.where(blk_causal[None], logits, _NEG)
    score = jax.nn.softmax(logits, axis=-1)
    diag = jnp.eye(NB, dtype=bool)
    mask = ((score >= THRESH) | diag[None]) & blk_causal[None]
    bias = jnp.where(mask, 0.0, _NEG).astype(_F32)
    bias_tok = jnp.repeat(bias, BLK, axis=2)
    return bias_tok


def _rope_tables(position_ids):
    inv_freq = 1.0 / (THETA ** (jnp.arange(0, HD, 2, dtype=_F32) / HD))
    freqs = position_ids[0].astype(_F32)[:, None] * inv_freq[None, :]
    emb = jnp.concatenate([freqs, freqs], axis=-1)
    return jnp.cos(emb), jnp.sin(emb)


def _rope(x, cos, sin):
    x1, x2 = jnp.split(x, 2, axis=-1)
    rot = jnp.concatenate([-x2, x1], axis=-1)
    return x * cos[:, None, :] + rot * sin[:, None, :]


def kernel(hidden_states, position_ids, ln1_w, q_w, q_b, k_w, k_b, v_w, v_b,
           o_w, gq_w, gk_w, ln2_w, gate_w, up_w, down_w):
    hs = hidden_states.reshape(S, D)

    cos, sin = _rope_tables(position_ids)
    xn1 = _rmsnorm(hs, ln1_w)
    bqkv = jnp.concatenate([q_b, k_b, v_b], axis=0)
    qkv, pooled = _qkv(xn1, q_w, k_w, v_w, bqkv, cos, sin)

    q_pool, k_pool = _gate_bias(pooled)
    bias_tok = _gate_mask_bias(q_pool, k_pool, gq_w, gk_w)
    bias5 = bias_tok.reshape(KVH, GQ, MQ, BQB, S)

    attn2 = _flash(qkv, bias5)

    hidden = _matmul_res(attn2, o_w, hs, bm=512, bn=1024)

    xn2 = _rmsnorm(hidden, ln2_w)
    mlp_mid = _mlp1(xn2, gate_w, up_w, bm=1024, bn=512)
    out = _matmul_res(mlp_mid, down_w, hidden, bm=512, bn=512)
    return out.reshape(1, S, D)
